# baseline (speedup 1.0000x reference)
import jax
import jax.numpy as jnp
from jax import lax
from jax.experimental import pallas as pl
from jax.experimental.pallas import tpu as pltpu
from jax.experimental.pallas import tpu_sc as plsc

PATCH = 196
EMBED = 768
BATCH = 256
LANES = 16
PAD = 208
NCHUNK = PAD // LANES
NC = 2
NS = 16
NW = NC * NS
B_PER_W = BATCH // NW
ROWS = BATCH * PATCH


def _sc_body(x_hbm, maskp_hbm, zeros_hbm, out_hbm, mask_v, keep_v, zero_v, gbuf, zbuf):
    wid = lax.axis_index("s") * NC + lax.axis_index("c")

    pltpu.sync_copy(maskp_hbm, mask_v)
    pltpu.sync_copy(zeros_hbm, zbuf)

    ones = jnp.full((LANES,), 1, jnp.int32)
    zeros16 = jnp.full((LANES,), 0, jnp.int32)
    nk_v = zeros16
    nz_v = zeros16
    for j in range(NCHUNK):
        m = mask_v[pl.ds(j * LANES, LANES)]
        p = j * LANES + lax.iota(jnp.int32, LANES)
        valid = p < PATCH
        keep = (m == 0) & (p > 0) & valid
        zero = jnp.logical_not(keep) & valid
        kpos = jnp.maximum(nk_v + lax.cumsum(jnp.where(keep, ones, zeros16)) - 1, 0)
        zpos = jnp.maximum(nz_v + lax.cumsum(jnp.where(zero, ones, zeros16)) - 1, 0)
        plsc.store_scatter(keep_v, [kpos], p, mask=keep)
        plsc.store_scatter(zero_v, [zpos], p, mask=zero)
        nk_v = nk_v + plsc.all_reduce_population_count(keep)
        nz_v = nz_v + plsc.all_reduce_population_count(zero)
    for j in range(NCHUNK):
        p = j * LANES + lax.iota(jnp.int32, LANES)
        plsc.store_scatter(keep_v, [p], zeros16, mask=p >= nk_v)
        plsc.store_scatter(zero_v, [p], zeros16, mask=p >= nz_v)
    n_keep = jnp.max(nk_v)
    n_zero = jnp.max(nz_v)

    def batch_body(bl, carry):
        b = wid * B_PER_W + bl
        base = jnp.full((LANES,), b * PATCH, jnp.int32)
        for j in range(NCHUNK):

            @pl.when(j * LANES < n_keep)
            def _():
                idx = keep_v[pl.ds(j * LANES, LANES)] + base
                pltpu.sync_copy(x_hbm.at[idx], gbuf)
                pltpu.sync_copy(gbuf, out_hbm.at[idx])

        for j in range(NCHUNK):

            @pl.when(j * LANES < n_zero)
            def _():
                zidx = zero_v[pl.ds(j * LANES, LANES)] + base
                pltpu.sync_copy(zbuf, out_hbm.at[zidx])

        return carry

    lax.fori_loop(0, B_PER_W, batch_body, 0)


def kernel(x, mask):
    x2d = x.reshape(ROWS, EMBED)
    maskp = jnp.concatenate(
        [mask.reshape(-1), jnp.ones((PAD - PATCH,), mask.dtype)]
    )
    zeros = jnp.zeros((LANES, EMBED), x.dtype)
    mesh = plsc.VectorSubcoreMesh(core_axis_name="c", subcore_axis_name="s")
    out = pl.kernel(
        _sc_body,
        out_type=jax.ShapeDtypeStruct((ROWS, EMBED), x.dtype),
        mesh=mesh,
        compiler_params=pltpu.CompilerParams(needs_layout_passes=False),
        scratch_types=[
            pltpu.VMEM((PAD,), jnp.int32),
            pltpu.VMEM((PAD,), jnp.int32),
            pltpu.VMEM((PAD,), jnp.int32),
            pltpu.VMEM((LANES, EMBED), jnp.float32),
            pltpu.VMEM((LANES, EMBED), jnp.float32),
        ],
    )(x2d, maskp, zeros)
    return (out.reshape(x.shape), mask)
